# baseline (speedup 1.0000x reference)
import functools

import jax
import jax.numpy as jnp
from jax import lax
from jax.experimental import pallas as pl
from jax.experimental.pallas import tpu as pltpu
from jax.experimental.pallas import tpu_sc as plsc

N = 10000
D = 128
E = 320000
BR = 512
NPAD = 10240
NW = 32
CHUNK = 128
NCHUNK = 80
EPAD = NW * NCHUNK * CHUNK
RPS = NPAD // 16

_mesh = plsc.VectorSubcoreMesh(core_axis_name="c", subcore_axis_name="s")


SB = 16
NPAIR = NCHUNK // 2


@functools.partial(
    pl.kernel,
    mesh=_mesh,
    out_type=jax.ShapeDtypeStruct((2, NPAD, D), jnp.float32),
    scratch_types=(
        [pltpu.VMEM((SB, CHUNK), jnp.int32)] * 2
        + [pltpu.VMEM((CHUNK, D), jnp.float32)] * 2
        + [pltpu.SemaphoreType.DMA] * 5
        + [pltpu.VMEM_SHARED((NPAD, D), jnp.float32)]
    ),
)
def _sc_aggregate(hp_hbm, src_hbm, dst_hbm, zeros_hbm, out_hbm,
                  src_sb, dst_sb, rows0, rows1,
                  isem, gsem0, gsem1, ssem0, ssem1, acc):
    c = lax.axis_index("c")
    s = lax.axis_index("s")
    w = c * 16 + s
    tile_row = w * NCHUNK
    pltpu.sync_copy(zeros_hbm, acc.at[pl.ds(s * RPS, RPS), :])
    pltpu.async_copy(src_hbm.at[pl.ds(tile_row, 8), :],
                     src_sb.at[pl.ds(0, 8), :], isem)
    pltpu.async_copy(dst_hbm.at[pl.ds(tile_row, 8), :],
                     dst_sb.at[pl.ds(0, 8), :], isem)
    plsc.subcore_barrier()

    slots = ((rows0, gsem0, ssem0), (rows1, gsem1, ssem1))

    def _drain_scatter(b):
        rows, _, ssem = slots[b]
        pltpu.make_async_copy(rows, acc.at[dst_sb.at[0]], ssem).wait()

    def _wait_isem():
        for sb in (src_sb, dst_sb):
            pltpu.make_async_copy(src_hbm.at[pl.ds(0, 8), :],
                                  sb.at[pl.ds(0, 8), :], isem).wait()

    def pair(t, carry):
        at_sup = lax.rem(t, 4) == 0

        @pl.when(at_sup & (t > 0))
        def _():
            _drain_scatter(0)
            _drain_scatter(1)

        @pl.when(at_sup)
        def _():
            _wait_isem()

        @pl.when(at_sup & (t < NPAIR - 4))
        def _():
            u1 = t // 4 + 1
            hrow = pl.multiple_of(tile_row + u1 * 8, 8)
            vrow = pl.multiple_of(lax.rem(u1, 2) * 8, 8)
            pltpu.async_copy(src_hbm.at[pl.ds(hrow, 8), :],
                             src_sb.at[pl.ds(vrow, 8), :], isem)
            pltpu.async_copy(dst_hbm.at[pl.ds(hrow, 8), :],
                             dst_sb.at[pl.ds(vrow, 8), :], isem)

        @pl.when(~at_sup)
        def _():
            _drain_scatter(0)
            _drain_scatter(1)

        for b in range(2):
            rows, gsem, _ = slots[b]
            r = lax.rem(2 * t + b, SB)
            pltpu.async_copy(hp_hbm.at[src_sb.at[r]], rows, gsem)
        for b in range(2):
            rows, gsem, ssem = slots[b]
            r = lax.rem(2 * t + b, SB)
            pltpu.make_async_copy(hp_hbm.at[src_sb.at[r]], rows, gsem).wait()
            pltpu.async_copy(rows, acc.at[dst_sb.at[r]], ssem, add=True)
        return carry

    lax.fori_loop(0, NPAIR, pair, 0)
    _drain_scatter(0)
    _drain_scatter(1)
    plsc.subcore_barrier()
    pltpu.sync_copy(acc.at[pl.ds(s * RPS, RPS), :],
                    out_hbm.at[c, pl.ds(s * RPS, RPS), :])


def _t1_body(x_ref, w_ref, degs_ref, hp_ref, dis_ref):
    i = pl.program_id(0)
    deg = degs_ref[0, :, 0:1] + degs_ref[1, :, 0:1] + 1.0
    row = lax.broadcasted_iota(jnp.int32, (BR, 1), 0) + i * BR
    dis = jnp.where(row < N, lax.rsqrt(deg), 0.0)
    h = jnp.dot(x_ref[...], w_ref[...], preferred_element_type=jnp.float32)
    hp_ref[...] = dis * h
    dis_ref[...] = dis


def _t1(x, w, degs):
    return pl.pallas_call(
        _t1_body,
        grid=(NPAD // BR,),
        in_specs=[
            pl.BlockSpec((BR, D), lambda i: (i, 0)),
            pl.BlockSpec((D, D), lambda i: (0, 0)),
            pl.BlockSpec((2, BR, D), lambda i: (0, i, 0)),
        ],
        out_specs=[
            pl.BlockSpec((BR, D), lambda i: (i, 0)),
            pl.BlockSpec((BR, 1), lambda i: (i, 0)),
        ],
        out_shape=[
            jax.ShapeDtypeStruct((NPAD, D), jnp.float32),
            jax.ShapeDtypeStruct((NPAD, 1), jnp.float32),
        ],
    )(x, w, degs)


def _tmid_body(p_ref, hp_ref, dis_ref, b_ref, w_ref, out_ref):
    dis = dis_ref[...]
    a = p_ref[0] + p_ref[1] + hp_ref[...]
    a = jnp.maximum(dis * a + b_ref[...], 0.0)
    out_ref[...] = dis * jnp.dot(a, w_ref[...],
                                 preferred_element_type=jnp.float32)


def _tmid(p, hp, dis, b, w):
    return pl.pallas_call(
        _tmid_body,
        grid=(NPAD // BR,),
        in_specs=[
            pl.BlockSpec((2, BR, D), lambda i: (0, i, 0)),
            pl.BlockSpec((BR, D), lambda i: (i, 0)),
            pl.BlockSpec((BR, 1), lambda i: (i, 0)),
            pl.BlockSpec((1, D), lambda i: (0, 0)),
            pl.BlockSpec((D, D), lambda i: (0, 0)),
        ],
        out_specs=pl.BlockSpec((BR, D), lambda i: (i, 0)),
        out_shape=jax.ShapeDtypeStruct((NPAD, D), jnp.float32),
    )(p, hp, dis, b, w)


def _t4_body(p_ref, hp_ref, dis_ref, b_ref, wfc_ref, bfc_ref, out_ref,
             acc_ref):
    i = pl.program_id(0)
    dis = dis_ref[...]
    a = p_ref[0] + p_ref[1] + hp_ref[...]
    a = jnp.maximum(dis * a + b_ref[...], 0.0)
    row = lax.broadcasted_iota(jnp.int32, (BR, 1), 0) + i * BR
    a = jnp.where(row < N, a, 0.0)
    part = jnp.sum(a, axis=0, keepdims=True)

    @pl.when(i == 0)
    def _():
        acc_ref[...] = part

    @pl.when(i > 0)
    def _():
        acc_ref[...] = acc_ref[...] + part

    @pl.when(i == pl.num_programs(0) - 1)
    def _():
        m = acc_ref[...] * (1.0 / N)
        logits = jnp.dot(m, wfc_ref[...],
                         preferred_element_type=jnp.float32) + bfc_ref[...]
        z = logits - jnp.max(logits, axis=-1, keepdims=True)
        e = jnp.exp(z)
        out_ref[...] = e / jnp.sum(e, axis=-1, keepdims=True)


def _t4(p, hp, dis, b, wfc, bfc):
    return pl.pallas_call(
        _t4_body,
        grid=(NPAD // BR,),
        in_specs=[
            pl.BlockSpec((2, BR, D), lambda i: (0, i, 0)),
            pl.BlockSpec((BR, D), lambda i: (i, 0)),
            pl.BlockSpec((BR, 1), lambda i: (i, 0)),
            pl.BlockSpec((1, D), lambda i: (0, 0)),
            pl.BlockSpec((D, 4), lambda i: (0, 0)),
            pl.BlockSpec((1, 4), lambda i: (0, 0)),
        ],
        out_specs=pl.BlockSpec((1, 4), lambda i: (0, 0)),
        out_shape=jax.ShapeDtypeStruct((1, 4), jnp.float32),
        scratch_shapes=[pltpu.VMEM((1, D), jnp.float32)],
    )(p, hp, dis, b, wfc, bfc)


def kernel(x, edge_index, W1, b1, W2, b2, W3, b3, Wfc, bfc):
    src = edge_index[0].astype(jnp.int32)
    dst = edge_index[1].astype(jnp.int32)
    pad = jnp.full((EPAD - E,), N, dtype=jnp.int32)
    src2d = jnp.concatenate([src, pad]).reshape(NW * NCHUNK, CHUNK)
    dst2d = jnp.concatenate([dst, pad]).reshape(NW * NCHUNK, CHUNK)
    xp = jnp.zeros((NPAD, D), jnp.float32).at[:N].set(x)

    onesM = jnp.ones((NPAD, D), jnp.float32)
    zerosD = jnp.zeros((RPS, D), jnp.float32)

    degs = _sc_aggregate(onesM, src2d, dst2d, zerosD)
    hp1, dis = _t1(xp, W1, degs)
    p1 = _sc_aggregate(hp1, src2d, dst2d, zerosD)
    hp2 = _tmid(p1, hp1, dis, b1.reshape(1, D), W2)
    p2 = _sc_aggregate(hp2, src2d, dst2d, zerosD)
    hp3 = _tmid(p2, hp2, dis, b2.reshape(1, D), W3)
    p3 = _sc_aggregate(hp3, src2d, dst2d, zerosD)
    return _t4(p3, hp3, dis, b3.reshape(1, D), Wfc, bfc.reshape(1, 4))


# speedup vs baseline: 1.2735x; 1.2735x over previous
import functools

import jax
import jax.numpy as jnp
from jax import lax
from jax.experimental import pallas as pl
from jax.experimental.pallas import tpu as pltpu
from jax.experimental.pallas import tpu_sc as plsc

N = 10000
D = 128
E = 320000
BR = 512
NPAD = 10240
NW = 32
CHUNK = 128
NCHUNK = 80
EPAD = NW * NCHUNK * CHUNK
RPS = NPAD // 16

_mesh = plsc.VectorSubcoreMesh(core_axis_name="c", subcore_axis_name="s")


@functools.partial(
    pl.kernel,
    mesh=_mesh,
    out_type=jax.ShapeDtypeStruct((2, NPAD, D), jnp.float32),
    scratch_types=[
        pltpu.VMEM((NCHUNK, CHUNK), jnp.int32),
        pltpu.VMEM((CHUNK, D), jnp.float32),
        pltpu.VMEM_SHARED((NPAD, D), jnp.float32),
    ],
)
def _sc_count(dst_hbm, ones_hbm, zeros_hbm, out_hbm, dst_v, ones_v, acc):
    c = lax.axis_index("c")
    s = lax.axis_index("s")
    w = c * 16 + s
    pltpu.sync_copy(zeros_hbm, acc.at[pl.ds(s * RPS, RPS), :])
    pltpu.sync_copy(ones_hbm, ones_v)
    pltpu.sync_copy(dst_hbm.at[pl.ds(w * NCHUNK, NCHUNK), :], dst_v)
    plsc.subcore_barrier()

    def body(j, carry):
        pltpu.sync_copy(ones_v, acc.at[dst_v.at[j]], add=True)
        return carry

    lax.fori_loop(0, NCHUNK, body, 0)
    plsc.subcore_barrier()
    pltpu.sync_copy(acc.at[pl.ds(s * RPS, RPS), :],
                    out_hbm.at[c, pl.ds(s * RPS, RPS), :])


SB = 16
NPAIR = NCHUNK // 2


@functools.partial(
    pl.kernel,
    mesh=_mesh,
    out_type=jax.ShapeDtypeStruct((2, NPAD, D), jnp.float32),
    scratch_types=(
        [pltpu.VMEM((SB, CHUNK), jnp.int32)] * 2
        + [pltpu.VMEM((CHUNK, D), jnp.float32)] * 2
        + [pltpu.SemaphoreType.DMA] * 5
        + [pltpu.VMEM_SHARED((NPAD, D), jnp.float32)]
    ),
)
def _sc_aggregate(hp_hbm, src_hbm, dst_hbm, zeros_hbm, out_hbm,
                  src_sb, dst_sb, rows0, rows1,
                  isem, gsem0, gsem1, ssem0, ssem1, acc):
    c = lax.axis_index("c")
    s = lax.axis_index("s")
    w = c * 16 + s
    tile_row = w * NCHUNK
    pltpu.sync_copy(zeros_hbm, acc.at[pl.ds(s * RPS, RPS), :])
    pltpu.async_copy(src_hbm.at[pl.ds(tile_row, 8), :],
                     src_sb.at[pl.ds(0, 8), :], isem)
    pltpu.async_copy(dst_hbm.at[pl.ds(tile_row, 8), :],
                     dst_sb.at[pl.ds(0, 8), :], isem)
    plsc.subcore_barrier()

    slots = ((rows0, gsem0, ssem0), (rows1, gsem1, ssem1))

    def _drain_scatter(b):
        rows, _, ssem = slots[b]
        pltpu.make_async_copy(rows, acc.at[dst_sb.at[0]], ssem).wait()

    def _wait_isem():
        for sb in (src_sb, dst_sb):
            pltpu.make_async_copy(src_hbm.at[pl.ds(0, 8), :],
                                  sb.at[pl.ds(0, 8), :], isem).wait()

    def pair(t, carry):
        at_sup = lax.rem(t, 4) == 0

        @pl.when(at_sup & (t > 0))
        def _():
            _drain_scatter(0)
            _drain_scatter(1)

        @pl.when(at_sup)
        def _():
            _wait_isem()

        @pl.when(at_sup & (t < NPAIR - 4))
        def _():
            u1 = t // 4 + 1
            hrow = pl.multiple_of(tile_row + u1 * 8, 8)
            vrow = pl.multiple_of(lax.rem(u1, 2) * 8, 8)
            pltpu.async_copy(src_hbm.at[pl.ds(hrow, 8), :],
                             src_sb.at[pl.ds(vrow, 8), :], isem)
            pltpu.async_copy(dst_hbm.at[pl.ds(hrow, 8), :],
                             dst_sb.at[pl.ds(vrow, 8), :], isem)

        @pl.when(~at_sup)
        def _():
            _drain_scatter(0)
            _drain_scatter(1)

        for b in range(2):
            rows, gsem, _ = slots[b]
            r = lax.rem(2 * t + b, SB)
            pltpu.async_copy(hp_hbm.at[src_sb.at[r]], rows, gsem)
        for b in range(2):
            rows, gsem, ssem = slots[b]
            r = lax.rem(2 * t + b, SB)
            pltpu.make_async_copy(hp_hbm.at[src_sb.at[r]], rows, gsem).wait()
            pltpu.async_copy(rows, acc.at[dst_sb.at[r]], ssem, add=True)
        return carry

    lax.fori_loop(0, NPAIR, pair, 0)
    _drain_scatter(0)
    _drain_scatter(1)
    plsc.subcore_barrier()
    pltpu.sync_copy(acc.at[pl.ds(s * RPS, RPS), :],
                    out_hbm.at[c, pl.ds(s * RPS, RPS), :])


def _t1_body(x_ref, w_ref, degs_ref, hp_ref, dis_ref):
    i = pl.program_id(0)
    deg = degs_ref[0, :, 0:1] + degs_ref[1, :, 0:1] + 1.0
    row = lax.broadcasted_iota(jnp.int32, (BR, 1), 0) + i * BR
    dis = jnp.where(row < N, lax.rsqrt(deg), 0.0)
    h = jnp.dot(x_ref[...], w_ref[...], preferred_element_type=jnp.float32)
    hp_ref[...] = dis * h
    dis_ref[...] = dis


def _t1(x, w, degs):
    return pl.pallas_call(
        _t1_body,
        grid=(NPAD // BR,),
        in_specs=[
            pl.BlockSpec((BR, D), lambda i: (i, 0)),
            pl.BlockSpec((D, D), lambda i: (0, 0)),
            pl.BlockSpec((2, BR, D), lambda i: (0, i, 0)),
        ],
        out_specs=[
            pl.BlockSpec((BR, D), lambda i: (i, 0)),
            pl.BlockSpec((BR, 1), lambda i: (i, 0)),
        ],
        out_shape=[
            jax.ShapeDtypeStruct((NPAD, D), jnp.float32),
            jax.ShapeDtypeStruct((NPAD, 1), jnp.float32),
        ],
    )(x, w, degs)


def _tmid_body(p_ref, hp_ref, dis_ref, b_ref, w_ref, out_ref):
    dis = dis_ref[...]
    a = p_ref[0] + p_ref[1] + hp_ref[...]
    a = jnp.maximum(dis * a + b_ref[...], 0.0)
    out_ref[...] = dis * jnp.dot(a, w_ref[...],
                                 preferred_element_type=jnp.float32)


def _tmid(p, hp, dis, b, w):
    return pl.pallas_call(
        _tmid_body,
        grid=(NPAD // BR,),
        in_specs=[
            pl.BlockSpec((2, BR, D), lambda i: (0, i, 0)),
            pl.BlockSpec((BR, D), lambda i: (i, 0)),
            pl.BlockSpec((BR, 1), lambda i: (i, 0)),
            pl.BlockSpec((1, D), lambda i: (0, 0)),
            pl.BlockSpec((D, D), lambda i: (0, 0)),
        ],
        out_specs=pl.BlockSpec((BR, D), lambda i: (i, 0)),
        out_shape=jax.ShapeDtypeStruct((NPAD, D), jnp.float32),
    )(p, hp, dis, b, w)


def _t4_body(p_ref, hp_ref, dis_ref, b_ref, wfc_ref, bfc_ref, out_ref,
             acc_ref):
    i = pl.program_id(0)
    dis = dis_ref[...]
    a = p_ref[0] + p_ref[1] + hp_ref[...]
    a = jnp.maximum(dis * a + b_ref[...], 0.0)
    row = lax.broadcasted_iota(jnp.int32, (BR, 1), 0) + i * BR
    a = jnp.where(row < N, a, 0.0)
    part = jnp.sum(a, axis=0, keepdims=True)

    @pl.when(i == 0)
    def _():
        acc_ref[...] = part

    @pl.when(i > 0)
    def _():
        acc_ref[...] = acc_ref[...] + part

    @pl.when(i == pl.num_programs(0) - 1)
    def _():
        m = acc_ref[...] * (1.0 / N)
        logits = jnp.dot(m, wfc_ref[...],
                         preferred_element_type=jnp.float32) + bfc_ref[...]
        z = logits - jnp.max(logits, axis=-1, keepdims=True)
        e = jnp.exp(z)
        out_ref[...] = e / jnp.sum(e, axis=-1, keepdims=True)


def _t4(p, hp, dis, b, wfc, bfc):
    return pl.pallas_call(
        _t4_body,
        grid=(NPAD // BR,),
        in_specs=[
            pl.BlockSpec((2, BR, D), lambda i: (0, i, 0)),
            pl.BlockSpec((BR, D), lambda i: (i, 0)),
            pl.BlockSpec((BR, 1), lambda i: (i, 0)),
            pl.BlockSpec((1, D), lambda i: (0, 0)),
            pl.BlockSpec((D, 4), lambda i: (0, 0)),
            pl.BlockSpec((1, 4), lambda i: (0, 0)),
        ],
        out_specs=pl.BlockSpec((1, 4), lambda i: (0, 0)),
        out_shape=jax.ShapeDtypeStruct((1, 4), jnp.float32),
        scratch_shapes=[pltpu.VMEM((1, D), jnp.float32)],
    )(p, hp, dis, b, wfc, bfc)


def kernel(x, edge_index, W1, b1, W2, b2, W3, b3, Wfc, bfc):
    src = edge_index[0].astype(jnp.int32)
    dst = edge_index[1].astype(jnp.int32)
    pad = jnp.full((EPAD - E,), N, dtype=jnp.int32)
    src2d = jnp.concatenate([src, pad]).reshape(NW * NCHUNK, CHUNK)
    dst2d = jnp.concatenate([dst, pad]).reshape(NW * NCHUNK, CHUNK)
    xp = jnp.zeros((NPAD, D), jnp.float32).at[:N].set(x)

    onesC = jnp.ones((CHUNK, D), jnp.float32)
    zerosD = jnp.zeros((RPS, D), jnp.float32)

    degs = _sc_count(dst2d, onesC, zerosD)
    hp1, dis = _t1(xp, W1, degs)
    p1 = _sc_aggregate(hp1, src2d, dst2d, zerosD)
    hp2 = _tmid(p1, hp1, dis, b1.reshape(1, D), W2)
    p2 = _sc_aggregate(hp2, src2d, dst2d, zerosD)
    hp3 = _tmid(p2, hp2, dis, b2.reshape(1, D), W3)
    p3 = _sc_aggregate(hp3, src2d, dst2d, zerosD)
    return _t4(p3, hp3, dis, b3.reshape(1, D), Wfc, bfc.reshape(1, 4))


# speedup vs baseline: 1.4378x; 1.1290x over previous
import functools

import jax
import jax.numpy as jnp
from jax import lax
from jax.experimental import pallas as pl
from jax.experimental.pallas import tpu as pltpu
from jax.experimental.pallas import tpu_sc as plsc

N = 10000
D = 128
E = 320000
BR = 512
NPAD = 10240
NW = 32
CHUNK = 128
NCHUNK = 80
EPAD = NW * NCHUNK * CHUNK
RPS = NPAD // 16

_mesh = plsc.VectorSubcoreMesh(core_axis_name="c", subcore_axis_name="s")


@functools.partial(
    pl.kernel,
    mesh=_mesh,
    out_type=jax.ShapeDtypeStruct((2, NPAD, D), jnp.float32),
    scratch_types=[
        pltpu.VMEM((NCHUNK, CHUNK), jnp.int32),
        pltpu.VMEM((CHUNK, D), jnp.float32),
        pltpu.VMEM_SHARED((NPAD, D), jnp.float32),
    ],
)
def _sc_count(dst_hbm, ones_hbm, zeros_hbm, out_hbm, dst_v, ones_v, acc):
    c = lax.axis_index("c")
    s = lax.axis_index("s")
    w = c * 16 + s
    pltpu.sync_copy(zeros_hbm, acc.at[pl.ds(s * RPS, RPS), :])
    pltpu.sync_copy(ones_hbm, ones_v)
    pltpu.sync_copy(dst_hbm.at[pl.ds(w * NCHUNK, NCHUNK), :], dst_v)
    plsc.subcore_barrier()

    def body(j, carry):
        pltpu.sync_copy(ones_v, acc.at[dst_v.at[j]], add=True)
        return carry

    lax.fori_loop(0, NCHUNK, body, 0)
    plsc.subcore_barrier()
    pltpu.sync_copy(acc.at[pl.ds(s * RPS, RPS), :],
                    out_hbm.at[c, pl.ds(s * RPS, RPS), :])


SB = 16
C0CH = 120
C1CH = 40


@functools.partial(
    pl.kernel,
    mesh=_mesh,
    out_type=jax.ShapeDtypeStruct((2, NPAD, D), jnp.float32),
    scratch_types=(
        [pltpu.VMEM((SB, CHUNK), jnp.int32)] * 2
        + [pltpu.VMEM((CHUNK, D), jnp.float32)] * 2
        + [pltpu.SemaphoreType.DMA] * 5
        + [pltpu.VMEM_SHARED((NPAD, D), jnp.float32)]
    ),
)
def _sc_aggregate(hp_hbm, src_hbm, dst_hbm, zeros_hbm, out_hbm,
                  src_sb, dst_sb, rows0, rows1,
                  isem, gsem0, gsem1, ssem0, ssem1, acc):
    c = lax.axis_index("c")
    s = lax.axis_index("s")
    nch = jnp.where(c == 0, C0CH, C1CH)
    tile_row = pl.multiple_of(
        jnp.where(c == 0, s * C0CH, 16 * C0CH + s * C1CH), 8)
    npair = nch // 2
    pltpu.sync_copy(zeros_hbm, acc.at[pl.ds(s * RPS, RPS), :])
    pltpu.async_copy(src_hbm.at[pl.ds(tile_row, 8), :],
                     src_sb.at[pl.ds(0, 8), :], isem)
    pltpu.async_copy(dst_hbm.at[pl.ds(tile_row, 8), :],
                     dst_sb.at[pl.ds(0, 8), :], isem)
    plsc.subcore_barrier()

    slots = ((rows0, gsem0, ssem0), (rows1, gsem1, ssem1))

    def _drain_scatter(b):
        rows, _, ssem = slots[b]
        pltpu.make_async_copy(rows, acc.at[dst_sb.at[0]], ssem).wait()

    def _wait_isem():
        for sb in (src_sb, dst_sb):
            pltpu.make_async_copy(src_hbm.at[pl.ds(0, 8), :],
                                  sb.at[pl.ds(0, 8), :], isem).wait()

    def pair(t, carry):
        at_sup = lax.rem(t, 4) == 0

        @pl.when(at_sup & (t > 0))
        def _():
            _drain_scatter(0)
            _drain_scatter(1)

        @pl.when(at_sup)
        def _():
            _wait_isem()

        @pl.when(at_sup & (t < npair - 4))
        def _():
            u1 = t // 4 + 1
            hrow = pl.multiple_of(tile_row + u1 * 8, 8)
            vrow = pl.multiple_of(lax.rem(u1, 2) * 8, 8)
            pltpu.async_copy(src_hbm.at[pl.ds(hrow, 8), :],
                             src_sb.at[pl.ds(vrow, 8), :], isem)
            pltpu.async_copy(dst_hbm.at[pl.ds(hrow, 8), :],
                             dst_sb.at[pl.ds(vrow, 8), :], isem)

        @pl.when(~at_sup)
        def _():
            _drain_scatter(0)
            _drain_scatter(1)

        for b in range(2):
            rows, gsem, _ = slots[b]
            r = lax.rem(2 * t + b, SB)
            pltpu.async_copy(hp_hbm.at[src_sb.at[r]], rows, gsem)
        for b in range(2):
            rows, gsem, ssem = slots[b]
            r = lax.rem(2 * t + b, SB)
            pltpu.make_async_copy(hp_hbm.at[src_sb.at[r]], rows, gsem).wait()
            pltpu.async_copy(rows, acc.at[dst_sb.at[r]], ssem, add=True)
        return carry

    lax.fori_loop(0, npair, pair, 0)
    _drain_scatter(0)
    _drain_scatter(1)
    plsc.subcore_barrier()
    pltpu.sync_copy(acc.at[pl.ds(s * RPS, RPS), :],
                    out_hbm.at[c, pl.ds(s * RPS, RPS), :])


def _t1_body(x_ref, w_ref, degs_ref, hp_ref, dis_ref):
    i = pl.program_id(0)
    deg = degs_ref[0, :, 0:1] + degs_ref[1, :, 0:1] + 1.0
    row = lax.broadcasted_iota(jnp.int32, (BR, 1), 0) + i * BR
    dis = jnp.where(row < N, lax.rsqrt(deg), 0.0)
    h = jnp.dot(x_ref[...], w_ref[...], preferred_element_type=jnp.float32)
    hp_ref[...] = dis * h
    dis_ref[...] = dis


def _t1(x, w, degs):
    return pl.pallas_call(
        _t1_body,
        grid=(NPAD // BR,),
        in_specs=[
            pl.BlockSpec((BR, D), lambda i: (i, 0)),
            pl.BlockSpec((D, D), lambda i: (0, 0)),
            pl.BlockSpec((2, BR, D), lambda i: (0, i, 0)),
        ],
        out_specs=[
            pl.BlockSpec((BR, D), lambda i: (i, 0)),
            pl.BlockSpec((BR, 1), lambda i: (i, 0)),
        ],
        out_shape=[
            jax.ShapeDtypeStruct((NPAD, D), jnp.float32),
            jax.ShapeDtypeStruct((NPAD, 1), jnp.float32),
        ],
    )(x, w, degs)


def _tmid_body(p_ref, hp_ref, dis_ref, b_ref, w_ref, out_ref):
    dis = dis_ref[...]
    a = p_ref[0] + p_ref[1] + hp_ref[...]
    a = jnp.maximum(dis * a + b_ref[...], 0.0)
    out_ref[...] = dis * jnp.dot(a, w_ref[...],
                                 preferred_element_type=jnp.float32)


def _tmid(p, hp, dis, b, w):
    return pl.pallas_call(
        _tmid_body,
        grid=(NPAD // BR,),
        in_specs=[
            pl.BlockSpec((2, BR, D), lambda i: (0, i, 0)),
            pl.BlockSpec((BR, D), lambda i: (i, 0)),
            pl.BlockSpec((BR, 1), lambda i: (i, 0)),
            pl.BlockSpec((1, D), lambda i: (0, 0)),
            pl.BlockSpec((D, D), lambda i: (0, 0)),
        ],
        out_specs=pl.BlockSpec((BR, D), lambda i: (i, 0)),
        out_shape=jax.ShapeDtypeStruct((NPAD, D), jnp.float32),
    )(p, hp, dis, b, w)


def _t4_body(p_ref, hp_ref, dis_ref, b_ref, wfc_ref, bfc_ref, out_ref,
             acc_ref):
    i = pl.program_id(0)
    dis = dis_ref[...]
    a = p_ref[0] + p_ref[1] + hp_ref[...]
    a = jnp.maximum(dis * a + b_ref[...], 0.0)
    row = lax.broadcasted_iota(jnp.int32, (BR, 1), 0) + i * BR
    a = jnp.where(row < N, a, 0.0)
    part = jnp.sum(a, axis=0, keepdims=True)

    @pl.when(i == 0)
    def _():
        acc_ref[...] = part

    @pl.when(i > 0)
    def _():
        acc_ref[...] = acc_ref[...] + part

    @pl.when(i == pl.num_programs(0) - 1)
    def _():
        m = acc_ref[...] * (1.0 / N)
        logits = jnp.dot(m, wfc_ref[...],
                         preferred_element_type=jnp.float32) + bfc_ref[...]
        z = logits - jnp.max(logits, axis=-1, keepdims=True)
        e = jnp.exp(z)
        out_ref[...] = e / jnp.sum(e, axis=-1, keepdims=True)


def _t4(p, hp, dis, b, wfc, bfc):
    return pl.pallas_call(
        _t4_body,
        grid=(NPAD // BR,),
        in_specs=[
            pl.BlockSpec((2, BR, D), lambda i: (0, i, 0)),
            pl.BlockSpec((BR, D), lambda i: (i, 0)),
            pl.BlockSpec((BR, 1), lambda i: (i, 0)),
            pl.BlockSpec((1, D), lambda i: (0, 0)),
            pl.BlockSpec((D, 4), lambda i: (0, 0)),
            pl.BlockSpec((1, 4), lambda i: (0, 0)),
        ],
        out_specs=pl.BlockSpec((1, 4), lambda i: (0, 0)),
        out_shape=jax.ShapeDtypeStruct((1, 4), jnp.float32),
        scratch_shapes=[pltpu.VMEM((1, D), jnp.float32)],
    )(p, hp, dis, b, wfc, bfc)


def kernel(x, edge_index, W1, b1, W2, b2, W3, b3, Wfc, bfc):
    src = edge_index[0].astype(jnp.int32)
    dst = edge_index[1].astype(jnp.int32)
    pad = jnp.full((EPAD - E,), N, dtype=jnp.int32)
    src2d = jnp.concatenate([src, pad]).reshape(NW * NCHUNK, CHUNK)
    dst2d = jnp.concatenate([dst, pad]).reshape(NW * NCHUNK, CHUNK)
    xp = jnp.zeros((NPAD, D), jnp.float32).at[:N].set(x)

    onesC = jnp.ones((CHUNK, D), jnp.float32)
    zerosD = jnp.zeros((RPS, D), jnp.float32)

    degs = _sc_count(dst2d, onesC, zerosD)
    hp1, dis = _t1(xp, W1, degs)
    p1 = _sc_aggregate(hp1, src2d, dst2d, zerosD)
    hp2 = _tmid(p1, hp1, dis, b1.reshape(1, D), W2)
    p2 = _sc_aggregate(hp2, src2d, dst2d, zerosD)
    hp3 = _tmid(p2, hp2, dis, b2.reshape(1, D), W3)
    p3 = _sc_aggregate(hp3, src2d, dst2d, zerosD)
    return _t4(p3, hp3, dis, b3.reshape(1, D), Wfc, bfc.reshape(1, 4))


# speedup vs baseline: 1.7242x; 1.1992x over previous
import functools

import jax
import jax.numpy as jnp
from jax import lax
from jax.experimental import pallas as pl
from jax.experimental.pallas import tpu as pltpu
from jax.experimental.pallas import tpu_sc as plsc

N = 10000
D = 128
E = 320000
BR = 512
NPAD = 10240
NW = 32
CHUNK = 128
NCHUNK = 80
EPAD = NW * NCHUNK * CHUNK
RPS = NPAD // 16

_mesh = plsc.VectorSubcoreMesh(core_axis_name="c", subcore_axis_name="s")


@functools.partial(
    pl.kernel,
    mesh=_mesh,
    out_type=jax.ShapeDtypeStruct((2, NPAD, D), jnp.float32),
    scratch_types=[
        pltpu.VMEM((NCHUNK, CHUNK), jnp.int32),
        pltpu.VMEM((CHUNK, D), jnp.float32),
        pltpu.VMEM_SHARED((NPAD, D), jnp.float32),
    ],
)
def _sc_count(dst_hbm, ones_hbm, zeros_hbm, out_hbm, dst_v, ones_v, acc):
    c = lax.axis_index("c")
    s = lax.axis_index("s")
    w = c * 16 + s
    pltpu.sync_copy(zeros_hbm, acc.at[pl.ds(s * RPS, RPS), :])
    pltpu.sync_copy(ones_hbm, ones_v)
    pltpu.sync_copy(dst_hbm.at[pl.ds(w * NCHUNK, NCHUNK), :], dst_v)
    plsc.subcore_barrier()

    def body(j, carry):
        pltpu.sync_copy(ones_v, acc.at[dst_v.at[j]], add=True)
        return carry

    lax.fori_loop(0, NCHUNK, body, 0)
    plsc.subcore_barrier()
    pltpu.sync_copy(acc.at[pl.ds(s * RPS, RPS), :],
                    out_hbm.at[c, pl.ds(s * RPS, RPS), :])


SB = 16
C0CH = 144
C1CH = 16


@functools.partial(
    pl.kernel,
    mesh=_mesh,
    out_type=jax.ShapeDtypeStruct((2, NPAD, D), jnp.float32),
    scratch_types=(
        [pltpu.VMEM((SB, CHUNK), jnp.int32)] * 2
        + [pltpu.VMEM((CHUNK, D), jnp.float32)] * 2
        + [pltpu.SemaphoreType.DMA] * 5
        + [pltpu.VMEM_SHARED((NPAD, D), jnp.float32)]
    ),
)
def _sc_aggregate(hp_hbm, src_hbm, dst_hbm, zeros_hbm, out_hbm,
                  src_sb, dst_sb, rows0, rows1,
                  isem, gsem0, gsem1, ssem0, ssem1, acc):
    c = lax.axis_index("c")
    s = lax.axis_index("s")
    nch = jnp.where(c == 0, C0CH, C1CH)
    tile_row = pl.multiple_of(
        jnp.where(c == 0, s * C0CH, 16 * C0CH + s * C1CH), 8)
    npair = nch // 2
    pltpu.sync_copy(zeros_hbm, acc.at[pl.ds(s * RPS, RPS), :])
    pltpu.async_copy(src_hbm.at[pl.ds(tile_row, 8), :],
                     src_sb.at[pl.ds(0, 8), :], isem)
    pltpu.async_copy(dst_hbm.at[pl.ds(tile_row, 8), :],
                     dst_sb.at[pl.ds(0, 8), :], isem)
    plsc.subcore_barrier()

    slots = ((rows0, gsem0, ssem0), (rows1, gsem1, ssem1))

    def _drain_scatter(b):
        rows, _, ssem = slots[b]
        pltpu.make_async_copy(rows, acc.at[dst_sb.at[0]], ssem).wait()

    def _wait_isem():
        for sb in (src_sb, dst_sb):
            pltpu.make_async_copy(src_hbm.at[pl.ds(0, 8), :],
                                  sb.at[pl.ds(0, 8), :], isem).wait()

    def pair(t, carry):
        at_sup = lax.rem(t, 4) == 0

        @pl.when(at_sup & (t > 0))
        def _():
            _drain_scatter(0)
            _drain_scatter(1)

        @pl.when(at_sup)
        def _():
            _wait_isem()

        @pl.when(at_sup & (t < npair - 4))
        def _():
            u1 = t // 4 + 1
            hrow = pl.multiple_of(tile_row + u1 * 8, 8)
            vrow = pl.multiple_of(lax.rem(u1, 2) * 8, 8)
            pltpu.async_copy(src_hbm.at[pl.ds(hrow, 8), :],
                             src_sb.at[pl.ds(vrow, 8), :], isem)
            pltpu.async_copy(dst_hbm.at[pl.ds(hrow, 8), :],
                             dst_sb.at[pl.ds(vrow, 8), :], isem)

        @pl.when(~at_sup)
        def _():
            _drain_scatter(0)
            _drain_scatter(1)

        for b in range(2):
            rows, gsem, _ = slots[b]
            r = lax.rem(2 * t + b, SB)
            pltpu.async_copy(hp_hbm.at[src_sb.at[r]], rows, gsem)
        for b in range(2):
            rows, gsem, ssem = slots[b]
            r = lax.rem(2 * t + b, SB)
            pltpu.make_async_copy(hp_hbm.at[src_sb.at[r]], rows, gsem).wait()
            pltpu.async_copy(rows, acc.at[dst_sb.at[r]], ssem, add=True)
        return carry

    lax.fori_loop(0, npair, pair, 0)
    _drain_scatter(0)
    _drain_scatter(1)
    plsc.subcore_barrier()
    pltpu.sync_copy(acc.at[pl.ds(s * RPS, RPS), :],
                    out_hbm.at[c, pl.ds(s * RPS, RPS), :])


def _t1_body(x_ref, w_ref, degs_ref, hp_ref, dis_ref):
    i = pl.program_id(0)
    deg = degs_ref[0, :, 0:1] + degs_ref[1, :, 0:1] + 1.0
    row = lax.broadcasted_iota(jnp.int32, (BR, 1), 0) + i * BR
    dis = jnp.where(row < N, lax.rsqrt(deg), 0.0)
    h = jnp.dot(x_ref[...], w_ref[...], preferred_element_type=jnp.float32)
    hp_ref[...] = dis * h
    dis_ref[...] = dis


def _t1(x, w, degs):
    return pl.pallas_call(
        _t1_body,
        grid=(NPAD // BR,),
        in_specs=[
            pl.BlockSpec((BR, D), lambda i: (i, 0)),
            pl.BlockSpec((D, D), lambda i: (0, 0)),
            pl.BlockSpec((2, BR, D), lambda i: (0, i, 0)),
        ],
        out_specs=[
            pl.BlockSpec((BR, D), lambda i: (i, 0)),
            pl.BlockSpec((BR, 1), lambda i: (i, 0)),
        ],
        out_shape=[
            jax.ShapeDtypeStruct((NPAD, D), jnp.float32),
            jax.ShapeDtypeStruct((NPAD, 1), jnp.float32),
        ],
    )(x, w, degs)


def _tmid_body(p_ref, hp_ref, dis_ref, b_ref, w_ref, out_ref):
    dis = dis_ref[...]
    a = p_ref[0] + p_ref[1] + hp_ref[...]
    a = jnp.maximum(dis * a + b_ref[...], 0.0)
    out_ref[...] = dis * jnp.dot(a, w_ref[...],
                                 preferred_element_type=jnp.float32)


def _tmid(p, hp, dis, b, w):
    return pl.pallas_call(
        _tmid_body,
        grid=(NPAD // BR,),
        in_specs=[
            pl.BlockSpec((2, BR, D), lambda i: (0, i, 0)),
            pl.BlockSpec((BR, D), lambda i: (i, 0)),
            pl.BlockSpec((BR, 1), lambda i: (i, 0)),
            pl.BlockSpec((1, D), lambda i: (0, 0)),
            pl.BlockSpec((D, D), lambda i: (0, 0)),
        ],
        out_specs=pl.BlockSpec((BR, D), lambda i: (i, 0)),
        out_shape=jax.ShapeDtypeStruct((NPAD, D), jnp.float32),
    )(p, hp, dis, b, w)


def _t4_body(p_ref, hp_ref, dis_ref, b_ref, wfc_ref, bfc_ref, out_ref,
             acc_ref):
    i = pl.program_id(0)
    dis = dis_ref[...]
    a = p_ref[0] + p_ref[1] + hp_ref[...]
    a = jnp.maximum(dis * a + b_ref[...], 0.0)
    row = lax.broadcasted_iota(jnp.int32, (BR, 1), 0) + i * BR
    a = jnp.where(row < N, a, 0.0)
    part = jnp.sum(a, axis=0, keepdims=True)

    @pl.when(i == 0)
    def _():
        acc_ref[...] = part

    @pl.when(i > 0)
    def _():
        acc_ref[...] = acc_ref[...] + part

    @pl.when(i == pl.num_programs(0) - 1)
    def _():
        m = acc_ref[...] * (1.0 / N)
        logits = jnp.dot(m, wfc_ref[...],
                         preferred_element_type=jnp.float32) + bfc_ref[...]
        z = logits - jnp.max(logits, axis=-1, keepdims=True)
        e = jnp.exp(z)
        out_ref[...] = e / jnp.sum(e, axis=-1, keepdims=True)


def _t4(p, hp, dis, b, wfc, bfc):
    return pl.pallas_call(
        _t4_body,
        grid=(NPAD // BR,),
        in_specs=[
            pl.BlockSpec((2, BR, D), lambda i: (0, i, 0)),
            pl.BlockSpec((BR, D), lambda i: (i, 0)),
            pl.BlockSpec((BR, 1), lambda i: (i, 0)),
            pl.BlockSpec((1, D), lambda i: (0, 0)),
            pl.BlockSpec((D, 4), lambda i: (0, 0)),
            pl.BlockSpec((1, 4), lambda i: (0, 0)),
        ],
        out_specs=pl.BlockSpec((1, 4), lambda i: (0, 0)),
        out_shape=jax.ShapeDtypeStruct((1, 4), jnp.float32),
        scratch_shapes=[pltpu.VMEM((1, D), jnp.float32)],
    )(p, hp, dis, b, wfc, bfc)


def kernel(x, edge_index, W1, b1, W2, b2, W3, b3, Wfc, bfc):
    src = edge_index[0].astype(jnp.int32)
    dst = edge_index[1].astype(jnp.int32)
    pad = jnp.full((EPAD - E,), N, dtype=jnp.int32)
    src2d = jnp.concatenate([src, pad]).reshape(NW * NCHUNK, CHUNK)
    dst2d = jnp.concatenate([dst, pad]).reshape(NW * NCHUNK, CHUNK)
    xp = jnp.zeros((NPAD, D), jnp.float32).at[:N].set(x)

    onesC = jnp.ones((CHUNK, D), jnp.float32)
    zerosD = jnp.zeros((RPS, D), jnp.float32)

    degs = _sc_count(dst2d, onesC, zerosD)
    hp1, dis = _t1(xp, W1, degs)
    p1 = _sc_aggregate(hp1, src2d, dst2d, zerosD)
    hp2 = _tmid(p1, hp1, dis, b1.reshape(1, D), W2)
    p2 = _sc_aggregate(hp2, src2d, dst2d, zerosD)
    hp3 = _tmid(p2, hp2, dis, b2.reshape(1, D), W3)
    p3 = _sc_aggregate(hp3, src2d, dst2d, zerosD)
    return _t4(p3, hp3, dis, b3.reshape(1, D), Wfc, bfc.reshape(1, 4))


# speedup vs baseline: 1.7486x; 1.0142x over previous
import functools

import jax
import jax.numpy as jnp
from jax import lax
from jax.experimental import pallas as pl
from jax.experimental.pallas import tpu as pltpu
from jax.experimental.pallas import tpu_sc as plsc

N = 10000
D = 128
E = 320000
BR = 512
NPAD = 10240
NW = 32
CHUNK = 128
NCHUNK = 80
EPAD = NW * NCHUNK * CHUNK
RPS = NPAD // 16

_mesh = plsc.VectorSubcoreMesh(core_axis_name="c", subcore_axis_name="s")


@functools.partial(
    pl.kernel,
    mesh=_mesh,
    out_type=jax.ShapeDtypeStruct((2, NPAD, D), jnp.float32),
    scratch_types=[
        pltpu.VMEM((NCHUNK, CHUNK), jnp.int32),
        pltpu.VMEM((CHUNK, D), jnp.float32),
        pltpu.VMEM_SHARED((NPAD, D), jnp.float32),
    ],
)
def _sc_count(dst_hbm, ones_hbm, zeros_hbm, out_hbm, dst_v, ones_v, acc):
    c = lax.axis_index("c")
    s = lax.axis_index("s")
    w = c * 16 + s
    pltpu.sync_copy(zeros_hbm, acc.at[pl.ds(s * RPS, RPS), :])
    pltpu.sync_copy(ones_hbm, ones_v)
    pltpu.sync_copy(dst_hbm.at[pl.ds(w * NCHUNK, NCHUNK), :], dst_v)
    plsc.subcore_barrier()

    def body(j, carry):
        pltpu.sync_copy(ones_v, acc.at[dst_v.at[j]], add=True)
        return carry

    lax.fori_loop(0, NCHUNK, body, 0)
    plsc.subcore_barrier()
    pltpu.sync_copy(acc.at[pl.ds(s * RPS, RPS), :],
                    out_hbm.at[c, pl.ds(s * RPS, RPS), :])


SB = 16
C0CH = 152
C1CH = 8


@functools.partial(
    pl.kernel,
    mesh=_mesh,
    out_type=jax.ShapeDtypeStruct((2, NPAD, D), jnp.float32),
    scratch_types=(
        [pltpu.VMEM((SB, CHUNK), jnp.int32)] * 2
        + [pltpu.VMEM((CHUNK, D), jnp.float32)] * 2
        + [pltpu.SemaphoreType.DMA] * 5
        + [pltpu.VMEM_SHARED((NPAD, D), jnp.float32)]
    ),
)
def _sc_aggregate(hp_hbm, src_hbm, dst_hbm, zeros_hbm, out_hbm,
                  src_sb, dst_sb, rows0, rows1,
                  isem, gsem0, gsem1, ssem0, ssem1, acc):
    c = lax.axis_index("c")
    s = lax.axis_index("s")
    nch = jnp.where(c == 0, C0CH, C1CH)
    tile_row = pl.multiple_of(
        jnp.where(c == 0, s * C0CH, 16 * C0CH + s * C1CH), 8)
    npair = nch // 2
    pltpu.sync_copy(zeros_hbm, acc.at[pl.ds(s * RPS, RPS), :])
    pltpu.async_copy(src_hbm.at[pl.ds(tile_row, 8), :],
                     src_sb.at[pl.ds(0, 8), :], isem)
    pltpu.async_copy(dst_hbm.at[pl.ds(tile_row, 8), :],
                     dst_sb.at[pl.ds(0, 8), :], isem)
    plsc.subcore_barrier()

    slots = ((rows0, gsem0, ssem0), (rows1, gsem1, ssem1))

    def _drain_scatter(b):
        rows, _, ssem = slots[b]
        pltpu.make_async_copy(rows, acc.at[dst_sb.at[0]], ssem).wait()

    def _wait_isem():
        for sb in (src_sb, dst_sb):
            pltpu.make_async_copy(src_hbm.at[pl.ds(0, 8), :],
                                  sb.at[pl.ds(0, 8), :], isem).wait()

    def pair(t, carry):
        at_sup = lax.rem(t, 4) == 0

        @pl.when(at_sup & (t > 0))
        def _():
            _drain_scatter(0)
            _drain_scatter(1)

        @pl.when(at_sup)
        def _():
            _wait_isem()

        @pl.when(at_sup & (t < npair - 4))
        def _():
            u1 = t // 4 + 1
            hrow = pl.multiple_of(tile_row + u1 * 8, 8)
            vrow = pl.multiple_of(lax.rem(u1, 2) * 8, 8)
            pltpu.async_copy(src_hbm.at[pl.ds(hrow, 8), :],
                             src_sb.at[pl.ds(vrow, 8), :], isem)
            pltpu.async_copy(dst_hbm.at[pl.ds(hrow, 8), :],
                             dst_sb.at[pl.ds(vrow, 8), :], isem)

        @pl.when(~at_sup)
        def _():
            _drain_scatter(0)
            _drain_scatter(1)

        for b in range(2):
            rows, gsem, _ = slots[b]
            r = lax.rem(2 * t + b, SB)
            pltpu.async_copy(hp_hbm.at[src_sb.at[r]], rows, gsem)
        for b in range(2):
            rows, gsem, ssem = slots[b]
            r = lax.rem(2 * t + b, SB)
            pltpu.make_async_copy(hp_hbm.at[src_sb.at[r]], rows, gsem).wait()
            pltpu.async_copy(rows, acc.at[dst_sb.at[r]], ssem, add=True)
        return carry

    lax.fori_loop(0, npair, pair, 0)
    _drain_scatter(0)
    _drain_scatter(1)
    plsc.subcore_barrier()
    pltpu.sync_copy(acc.at[pl.ds(s * RPS, RPS), :],
                    out_hbm.at[c, pl.ds(s * RPS, RPS), :])


def _t1_body(x_ref, w_ref, degs_ref, hp_ref, dis_ref):
    i = pl.program_id(0)
    deg = degs_ref[0, :, 0:1] + degs_ref[1, :, 0:1] + 1.0
    row = lax.broadcasted_iota(jnp.int32, (BR, 1), 0) + i * BR
    dis = jnp.where(row < N, lax.rsqrt(deg), 0.0)
    h = jnp.dot(x_ref[...], w_ref[...], preferred_element_type=jnp.float32)
    hp_ref[...] = dis * h
    dis_ref[...] = dis


def _t1(x, w, degs):
    return pl.pallas_call(
        _t1_body,
        grid=(NPAD // BR,),
        in_specs=[
            pl.BlockSpec((BR, D), lambda i: (i, 0)),
            pl.BlockSpec((D, D), lambda i: (0, 0)),
            pl.BlockSpec((2, BR, D), lambda i: (0, i, 0)),
        ],
        out_specs=[
            pl.BlockSpec((BR, D), lambda i: (i, 0)),
            pl.BlockSpec((BR, 1), lambda i: (i, 0)),
        ],
        out_shape=[
            jax.ShapeDtypeStruct((NPAD, D), jnp.float32),
            jax.ShapeDtypeStruct((NPAD, 1), jnp.float32),
        ],
    )(x, w, degs)


def _tmid_body(p_ref, hp_ref, dis_ref, b_ref, w_ref, out_ref):
    dis = dis_ref[...]
    a = p_ref[0] + p_ref[1] + hp_ref[...]
    a = jnp.maximum(dis * a + b_ref[...], 0.0)
    out_ref[...] = dis * jnp.dot(a, w_ref[...],
                                 preferred_element_type=jnp.float32)


def _tmid(p, hp, dis, b, w):
    return pl.pallas_call(
        _tmid_body,
        grid=(NPAD // BR,),
        in_specs=[
            pl.BlockSpec((2, BR, D), lambda i: (0, i, 0)),
            pl.BlockSpec((BR, D), lambda i: (i, 0)),
            pl.BlockSpec((BR, 1), lambda i: (i, 0)),
            pl.BlockSpec((1, D), lambda i: (0, 0)),
            pl.BlockSpec((D, D), lambda i: (0, 0)),
        ],
        out_specs=pl.BlockSpec((BR, D), lambda i: (i, 0)),
        out_shape=jax.ShapeDtypeStruct((NPAD, D), jnp.float32),
    )(p, hp, dis, b, w)


def _t4_body(p_ref, hp_ref, dis_ref, b_ref, wfc_ref, bfc_ref, out_ref,
             acc_ref):
    i = pl.program_id(0)
    dis = dis_ref[...]
    a = p_ref[0] + p_ref[1] + hp_ref[...]
    a = jnp.maximum(dis * a + b_ref[...], 0.0)
    row = lax.broadcasted_iota(jnp.int32, (BR, 1), 0) + i * BR
    a = jnp.where(row < N, a, 0.0)
    part = jnp.sum(a, axis=0, keepdims=True)

    @pl.when(i == 0)
    def _():
        acc_ref[...] = part

    @pl.when(i > 0)
    def _():
        acc_ref[...] = acc_ref[...] + part

    @pl.when(i == pl.num_programs(0) - 1)
    def _():
        m = acc_ref[...] * (1.0 / N)
        logits = jnp.dot(m, wfc_ref[...],
                         preferred_element_type=jnp.float32) + bfc_ref[...]
        z = logits - jnp.max(logits, axis=-1, keepdims=True)
        e = jnp.exp(z)
        out_ref[...] = e / jnp.sum(e, axis=-1, keepdims=True)


def _t4(p, hp, dis, b, wfc, bfc):
    return pl.pallas_call(
        _t4_body,
        grid=(NPAD // BR,),
        in_specs=[
            pl.BlockSpec((2, BR, D), lambda i: (0, i, 0)),
            pl.BlockSpec((BR, D), lambda i: (i, 0)),
            pl.BlockSpec((BR, 1), lambda i: (i, 0)),
            pl.BlockSpec((1, D), lambda i: (0, 0)),
            pl.BlockSpec((D, 4), lambda i: (0, 0)),
            pl.BlockSpec((1, 4), lambda i: (0, 0)),
        ],
        out_specs=pl.BlockSpec((1, 4), lambda i: (0, 0)),
        out_shape=jax.ShapeDtypeStruct((1, 4), jnp.float32),
        scratch_shapes=[pltpu.VMEM((1, D), jnp.float32)],
    )(p, hp, dis, b, wfc, bfc)


def kernel(x, edge_index, W1, b1, W2, b2, W3, b3, Wfc, bfc):
    src = edge_index[0].astype(jnp.int32)
    dst = edge_index[1].astype(jnp.int32)
    pad = jnp.full((EPAD - E,), N, dtype=jnp.int32)
    src2d = jnp.concatenate([src, pad]).reshape(NW * NCHUNK, CHUNK)
    dst2d = jnp.concatenate([dst, pad]).reshape(NW * NCHUNK, CHUNK)
    xp = jnp.zeros((NPAD, D), jnp.float32).at[:N].set(x)

    onesC = jnp.ones((CHUNK, D), jnp.float32)
    zerosD = jnp.zeros((RPS, D), jnp.float32)

    degs = _sc_count(dst2d, onesC, zerosD)
    hp1, dis = _t1(xp, W1, degs)
    p1 = _sc_aggregate(hp1, src2d, dst2d, zerosD)
    hp2 = _tmid(p1, hp1, dis, b1.reshape(1, D), W2)
    p2 = _sc_aggregate(hp2, src2d, dst2d, zerosD)
    hp3 = _tmid(p2, hp2, dis, b2.reshape(1, D), W3)
    p3 = _sc_aggregate(hp3, src2d, dst2d, zerosD)
    return _t4(p3, hp3, dis, b3.reshape(1, D), Wfc, bfc.reshape(1, 4))
